# Initial kernel scaffold; baseline (speedup 1.0000x reference)
import jax
import jax.numpy as jnp
from jax.experimental import pallas as pl


def kernel(x, gate_w, w_gate_proj, w_up_proj, w_down_proj):
    raise NotImplementedError("write your pallas kernel here")


# speedup vs baseline: 2.3889x; 2.3889x over previous
import functools

import jax
import jax.numpy as jnp
from jax import lax
from jax.experimental import pallas as pl
from jax.experimental.pallas import tpu as pltpu
from jax.experimental.pallas import tpu_sc as plsc

HIDDEN = 1024
INTER = 1024
E = 8
T = 2048
BT = 256
NB = T // BT + E - 1
NBPAD = 16
NBUF = NB * BT
NW = 32
TPW = T // NW


def _router_body(x_ref, gw_ref, pos_ref, pval_ref, be_ref):
    x = x_ref[...]
    gw = gw_ref[...]
    logits = lax.dot_general(
        x, gw, (((1,), (1,)), ((), ())),
        preferred_element_type=jnp.float32,
        precision=lax.Precision.DEFAULT)
    lmax = jnp.max(logits, axis=1, keepdims=True)
    p = jnp.exp(logits - lmax)
    psum = jnp.sum(p, axis=1, keepdims=True)
    pmax = jnp.max(p, axis=1, keepdims=True)
    e_iota = lax.broadcasted_iota(jnp.int32, (T, E), 1)
    eid = jnp.min(jnp.where(p == pmax, e_iota, E), axis=1, keepdims=True)
    pval_ref[...] = pmax / psum

    oh = (e_iota == eid).astype(jnp.bfloat16)
    r_i = lax.broadcasted_iota(jnp.int32, (BT, BT), 0)
    c_i = lax.broadcasted_iota(jnp.int32, (BT, BT), 1)
    tri = (c_i < r_i).astype(jnp.bfloat16)
    rank_chunks = []
    running = jnp.zeros((1, E), jnp.float32)
    for i in range(T // BT):
        ohc = oh[i * BT:(i + 1) * BT]
        within = lax.dot_general(
            tri, ohc, (((1,), (0,)), ((), ())),
            preferred_element_type=jnp.float32)
        rank_chunks.append(within + running)
        running = running + jnp.sum(ohc.astype(jnp.float32), axis=0,
                                    keepdims=True)
    rank_all = jnp.concatenate(rank_chunks, axis=0)
    ohf = oh.astype(jnp.float32)
    rank = jnp.sum(rank_all * ohf, axis=1, keepdims=True)

    counts = running
    nb = jnp.floor((counts + (BT - 1)) / BT)
    u_r = lax.broadcasted_iota(jnp.int32, (E, E), 0)
    u_c = lax.broadcasted_iota(jnp.int32, (E, E), 1)
    u_incl = (u_r <= u_c).astype(jnp.float32)
    ic = lax.dot_general(
        nb, u_incl, (((1,), (0,)), ((), ())),
        preferred_element_type=jnp.float32)
    offs = (ic - nb) * BT
    pos = jnp.sum(ohf * offs, axis=1, keepdims=True) + rank
    pos_ref[...] = pos.astype(jnp.int32)

    b_iota = lax.broadcasted_iota(jnp.int32, (NBPAD, E), 0).astype(jnp.float32)
    be = jnp.sum((jnp.broadcast_to(ic, (NBPAD, E)) <= b_iota)
                 .astype(jnp.int32), axis=1, keepdims=True)
    be_ref[...] = jnp.minimum(be, E - 1)


_router = pl.pallas_call(
    _router_body,
    out_shape=(
        jax.ShapeDtypeStruct((T, 1), jnp.int32),
        jax.ShapeDtypeStruct((T, 1), jnp.float32),
        jax.ShapeDtypeStruct((NBPAD, 1), jnp.int32),
    ),
)


@functools.lru_cache(maxsize=None)
def _sc_kernels():
    mesh = plsc.VectorSubcoreMesh(core_axis_name="c", subcore_axis_name="s")

    @functools.partial(
        pl.kernel,
        out_type=jax.ShapeDtypeStruct((NBUF, HIDDEN), jnp.float32),
        scratch_types=[
            pltpu.VMEM((TPW,), jnp.int32),
            pltpu.VMEM((TPW, HIDDEN), jnp.float32),
            pltpu.SemaphoreType.DMA,
        ],
        mesh=mesh,
    )
    def dispatch(x_hbm, pos_hbm, xs_hbm, idx_v, rows_v, sem):
        wid = lax.axis_index("s") * 2 + lax.axis_index("c")
        base = wid * TPW
        pltpu.sync_copy(pos_hbm.at[pl.ds(base, TPW)], idx_v)
        pltpu.sync_copy(x_hbm.at[pl.ds(base, TPW)], rows_v)
        pltpu.async_copy(rows_v, xs_hbm.at[idx_v], sem).wait()

    @functools.partial(
        pl.kernel,
        out_type=jax.ShapeDtypeStruct((T, HIDDEN), jnp.float32),
        scratch_types=[
            pltpu.VMEM((TPW,), jnp.int32),
            pltpu.VMEM((TPW, HIDDEN), jnp.float32),
            pltpu.SemaphoreType.DMA,
        ],
        mesh=mesh,
    )
    def combine(ys_hbm, pos_hbm, out_hbm, idx_v, rows_v, sem):
        wid = lax.axis_index("s") * 2 + lax.axis_index("c")
        base = wid * TPW
        pltpu.sync_copy(pos_hbm.at[pl.ds(base, TPW)], idx_v)
        pltpu.async_copy(ys_hbm.at[idx_v], rows_v, sem).wait()
        pltpu.sync_copy(rows_v, out_hbm.at[pl.ds(base, TPW)])

    return dispatch, combine


def _expert_body(be_ref, xs_ref, wg_ref, wu_ref, wd_ref, ys_ref):
    xb = xs_ref[...].astype(jnp.bfloat16)
    wg = wg_ref[0].astype(jnp.bfloat16)
    wu = wu_ref[0].astype(jnp.bfloat16)
    wd = wd_ref[0].astype(jnp.bfloat16)
    g = lax.dot_general(xb, wg, (((1,), (1,)), ((), ())),
                        preferred_element_type=jnp.float32)
    u = lax.dot_general(xb, wu, (((1,), (1,)), ((), ())),
                        preferred_element_type=jnp.float32)
    h = (g / (1.0 + jnp.exp(-g))) * u
    ys_ref[...] = lax.dot_general(
        h.astype(jnp.bfloat16), wd, (((1,), (1,)), ((), ())),
        preferred_element_type=jnp.float32)


_experts = pl.pallas_call(
    _expert_body,
    grid_spec=pltpu.PrefetchScalarGridSpec(
        num_scalar_prefetch=1,
        grid=(NB,),
        in_specs=[
            pl.BlockSpec((BT, HIDDEN), lambda b, be: (b, 0)),
            pl.BlockSpec((1, INTER, HIDDEN), lambda b, be: (be[b], 0, 0)),
            pl.BlockSpec((1, INTER, HIDDEN), lambda b, be: (be[b], 0, 0)),
            pl.BlockSpec((1, HIDDEN, INTER), lambda b, be: (be[b], 0, 0)),
        ],
        out_specs=pl.BlockSpec((BT, HIDDEN), lambda b, be: (b, 0)),
    ),
    out_shape=jax.ShapeDtypeStruct((NBUF, HIDDEN), jnp.float32),
)


def _scale_body(y_ref, p_ref, o_ref):
    o_ref[...] = y_ref[...] * p_ref[...]


_scale = pl.pallas_call(
    _scale_body,
    grid=(T // BT,),
    in_specs=[
        pl.BlockSpec((BT, HIDDEN), lambda i: (i, 0)),
        pl.BlockSpec((BT, 1), lambda i: (i, 0)),
    ],
    out_specs=pl.BlockSpec((BT, HIDDEN), lambda i: (i, 0)),
    out_shape=jax.ShapeDtypeStruct((T, HIDDEN), jnp.float32),
)


def kernel(x, gate_w, w_gate_proj, w_up_proj, w_down_proj):
    pos2d, pval2d, be2d = _router(x, gate_w)
    pos = pos2d.reshape(T)
    be = be2d.reshape(NBPAD)
    dispatch, combine = _sc_kernels()
    xs = dispatch(x, pos)
    ys = _experts(be, xs, w_gate_proj, w_up_proj, w_down_proj)
    yt = combine(ys, pos)
    return _scale(yt, pval2d)


# speedup vs baseline: 2.4377x; 1.0204x over previous
import functools

import jax
import jax.numpy as jnp
from jax import lax
from jax.experimental import pallas as pl
from jax.experimental.pallas import tpu as pltpu
from jax.experimental.pallas import tpu_sc as plsc

HIDDEN = 1024
INTER = 1024
E = 8
T = 2048
BT = 256
NB = T // BT + E - 1
NBPAD = 16
NBUF = NB * BT
NW = 32
TPW = T // NW


def _router_body(x_ref, gw_ref, pos_ref, pval_ref, be_ref, used_ref):
    x = x_ref[...]
    gw = gw_ref[...]
    logits = lax.dot_general(
        x, gw, (((1,), (1,)), ((), ())),
        preferred_element_type=jnp.float32,
        precision=lax.Precision.DEFAULT)
    lmax = jnp.max(logits, axis=1, keepdims=True)
    p = jnp.exp(logits - lmax)
    psum = jnp.sum(p, axis=1, keepdims=True)
    pmax = jnp.max(p, axis=1, keepdims=True)
    e_iota = lax.broadcasted_iota(jnp.int32, (T, E), 1)
    eid = jnp.min(jnp.where(p == pmax, e_iota, E), axis=1, keepdims=True)
    pval_ref[...] = pmax / psum

    oh = (e_iota == eid).astype(jnp.bfloat16)
    r_i = lax.broadcasted_iota(jnp.int32, (BT, BT), 0)
    c_i = lax.broadcasted_iota(jnp.int32, (BT, BT), 1)
    tri = (c_i < r_i).astype(jnp.bfloat16)
    rank_chunks = []
    running = jnp.zeros((1, E), jnp.float32)
    for i in range(T // BT):
        ohc = oh[i * BT:(i + 1) * BT]
        within = lax.dot_general(
            tri, ohc, (((1,), (0,)), ((), ())),
            preferred_element_type=jnp.float32)
        rank_chunks.append(within + running)
        running = running + jnp.sum(ohc.astype(jnp.float32), axis=0,
                                    keepdims=True)
    rank_all = jnp.concatenate(rank_chunks, axis=0)
    ohf = oh.astype(jnp.float32)
    rank = jnp.sum(rank_all * ohf, axis=1, keepdims=True)

    counts = running
    nb = jnp.floor((counts + (BT - 1)) / BT)
    u_r = lax.broadcasted_iota(jnp.int32, (E, E), 0)
    u_c = lax.broadcasted_iota(jnp.int32, (E, E), 1)
    u_incl = (u_r <= u_c).astype(jnp.float32)
    ic = lax.dot_general(
        nb, u_incl, (((1,), (0,)), ((), ())),
        preferred_element_type=jnp.float32)
    offs = (ic - nb) * BT
    pos = jnp.sum(ohf * offs, axis=1, keepdims=True) + rank
    pos_ref[...] = pos.astype(jnp.int32)

    b_iota = lax.broadcasted_iota(jnp.int32, (NBPAD, E), 0).astype(jnp.float32)
    be = jnp.sum((jnp.broadcast_to(ic, (NBPAD, E)) <= b_iota)
                 .astype(jnp.int32), axis=1, keepdims=True)
    be_ref[...] = jnp.minimum(be, E - 1)
    used_ref[...] = ic[:, E - 1:E].astype(jnp.int32)


_router = pl.pallas_call(
    _router_body,
    out_shape=(
        jax.ShapeDtypeStruct((T, 1), jnp.int32),
        jax.ShapeDtypeStruct((T, 1), jnp.float32),
        jax.ShapeDtypeStruct((NBPAD, 1), jnp.int32),
        jax.ShapeDtypeStruct((1, 1), jnp.int32),
    ),
)


@functools.lru_cache(maxsize=None)
def _sc_kernels():
    mesh = plsc.VectorSubcoreMesh(core_axis_name="c", subcore_axis_name="s")

    @functools.partial(
        pl.kernel,
        out_type=jax.ShapeDtypeStruct((NBUF, HIDDEN), jnp.float32),
        scratch_types=[
            pltpu.VMEM((TPW,), jnp.int32),
            pltpu.VMEM((TPW, HIDDEN), jnp.float32),
            pltpu.SemaphoreType.DMA,
        ],
        mesh=mesh,
    )
    def dispatch(x_hbm, pos_hbm, xs_hbm, idx_v, rows_v, sem):
        wid = lax.axis_index("s") * 2 + lax.axis_index("c")
        base = wid * TPW
        pltpu.sync_copy(pos_hbm.at[pl.ds(base, TPW)], idx_v)
        pltpu.sync_copy(x_hbm.at[pl.ds(base, TPW)], rows_v)
        pltpu.async_copy(rows_v, xs_hbm.at[idx_v], sem).wait()

    @functools.partial(
        pl.kernel,
        out_type=jax.ShapeDtypeStruct((T, HIDDEN), jnp.float32),
        scratch_types=[
            pltpu.VMEM((TPW,), jnp.int32),
            pltpu.VMEM((TPW, HIDDEN), jnp.float32),
            pltpu.SemaphoreType.DMA,
        ],
        mesh=mesh,
    )
    def combine(ys_hbm, pos_hbm, out_hbm, idx_v, rows_v, sem):
        wid = lax.axis_index("s") * 2 + lax.axis_index("c")
        base = wid * TPW
        pltpu.sync_copy(pos_hbm.at[pl.ds(base, TPW)], idx_v)
        pltpu.async_copy(ys_hbm.at[idx_v], rows_v, sem).wait()
        pltpu.sync_copy(rows_v, out_hbm.at[pl.ds(base, TPW)])

    return dispatch, combine


def _expert_body(be_ref, used_ref, xs_ref, wg_ref, wu_ref, wd_ref, ys_ref,
                 wg_bf, wu_bf, wd_bf):
    b = pl.program_id(0)
    used = used_ref[0]
    live = b < used
    first = jnp.logical_and(
        live, jnp.logical_or(b == 0, be_ref[b] != be_ref[jnp.maximum(b - 1, 0)]))

    @pl.when(first)
    def _cast_weights():
        wg_bf[...] = wg_ref[0].astype(jnp.bfloat16)
        wu_bf[...] = wu_ref[0].astype(jnp.bfloat16)
        wd_bf[...] = wd_ref[0].astype(jnp.bfloat16)

    @pl.when(live)
    def _compute():
        xb = xs_ref[...].astype(jnp.bfloat16)
        g = lax.dot_general(xb, wg_bf[...], (((1,), (1,)), ((), ())),
                            preferred_element_type=jnp.float32)
        u = lax.dot_general(xb, wu_bf[...], (((1,), (1,)), ((), ())),
                            preferred_element_type=jnp.float32)
        h = (g / (1.0 + jnp.exp(-g))) * u
        ys_ref[...] = lax.dot_general(
            h.astype(jnp.bfloat16), wd_bf[...], (((1,), (1,)), ((), ())),
            preferred_element_type=jnp.float32)


def _w_idx(b, be, used):
    return be[jnp.minimum(b, used[0] - 1)]


_experts = pl.pallas_call(
    _expert_body,
    grid_spec=pltpu.PrefetchScalarGridSpec(
        num_scalar_prefetch=2,
        grid=(NB,),
        in_specs=[
            pl.BlockSpec((BT, HIDDEN),
                         lambda b, be, used: (jnp.minimum(b, used[0] - 1), 0)),
            pl.BlockSpec((1, INTER, HIDDEN), lambda b, be, used: (_w_idx(b, be, used), 0, 0)),
            pl.BlockSpec((1, INTER, HIDDEN), lambda b, be, used: (_w_idx(b, be, used), 0, 0)),
            pl.BlockSpec((1, HIDDEN, INTER), lambda b, be, used: (_w_idx(b, be, used), 0, 0)),
        ],
        out_specs=pl.BlockSpec((BT, HIDDEN), lambda b, be, used: (b, 0)),
        scratch_shapes=[
            pltpu.VMEM((INTER, HIDDEN), jnp.bfloat16),
            pltpu.VMEM((INTER, HIDDEN), jnp.bfloat16),
            pltpu.VMEM((HIDDEN, INTER), jnp.bfloat16),
        ],
    ),
    out_shape=jax.ShapeDtypeStruct((NBUF, HIDDEN), jnp.float32),
)


def _scale_body(y_ref, p_ref, o_ref):
    o_ref[...] = y_ref[...] * p_ref[...]


_scale = pl.pallas_call(
    _scale_body,
    grid=(T // BT,),
    in_specs=[
        pl.BlockSpec((BT, HIDDEN), lambda i: (i, 0)),
        pl.BlockSpec((BT, 1), lambda i: (i, 0)),
    ],
    out_specs=pl.BlockSpec((BT, HIDDEN), lambda i: (i, 0)),
    out_shape=jax.ShapeDtypeStruct((T, HIDDEN), jnp.float32),
)


def kernel(x, gate_w, w_gate_proj, w_up_proj, w_down_proj):
    pos2d, pval2d, be2d, used2d = _router(x, gate_w)
    pos = pos2d.reshape(T)
    be = be2d.reshape(NBPAD)
    used = used2d.reshape(1)
    dispatch, combine = _sc_kernels()
    xs = dispatch(x, pos)
    ys = _experts(be, used, xs, w_gate_proj, w_up_proj, w_down_proj)
    yt = combine(ys, pos)
    return _scale(yt, pval2d)


# speedup vs baseline: 2.6063x; 1.0692x over previous
import functools

import jax
import jax.numpy as jnp
from jax import lax
from jax.experimental import pallas as pl
from jax.experimental.pallas import tpu as pltpu
from jax.experimental.pallas import tpu_sc as plsc

HIDDEN = 1024
INTER = 1024
E = 8
T = 2048
BT = 256
NB = T // BT + E - 1
NBPAD = 16
NBUF = NB * BT
NW = 32
TPW = T // NW


def _router_body(x_ref, gw_ref, pos_ref, pval_ref, be_ref, used_ref,
                 sl_ref, nxt_ref):
    x = x_ref[...]
    gw = gw_ref[...]
    logits = lax.dot_general(
        x, gw, (((1,), (1,)), ((), ())),
        preferred_element_type=jnp.float32,
        precision=lax.Precision.DEFAULT)
    lmax = jnp.max(logits, axis=1, keepdims=True)
    p = jnp.exp(logits - lmax)
    psum = jnp.sum(p, axis=1, keepdims=True)
    pmax = jnp.max(p, axis=1, keepdims=True)
    e_iota = lax.broadcasted_iota(jnp.int32, (T, E), 1)
    eid = jnp.min(jnp.where(p == pmax, e_iota, E), axis=1, keepdims=True)
    pval_ref[...] = pmax / psum

    oh = (e_iota == eid).astype(jnp.bfloat16)
    r_i = lax.broadcasted_iota(jnp.int32, (BT, BT), 0)
    c_i = lax.broadcasted_iota(jnp.int32, (BT, BT), 1)
    tri = (c_i < r_i).astype(jnp.bfloat16)
    rank_chunks = []
    running = jnp.zeros((1, E), jnp.float32)
    for i in range(T // BT):
        ohc = oh[i * BT:(i + 1) * BT]
        within = lax.dot_general(
            tri, ohc, (((1,), (0,)), ((), ())),
            preferred_element_type=jnp.float32)
        rank_chunks.append(within + running)
        running = running + jnp.sum(ohc.astype(jnp.float32), axis=0,
                                    keepdims=True)
    rank_all = jnp.concatenate(rank_chunks, axis=0)
    ohf = oh.astype(jnp.float32)
    rank = jnp.sum(rank_all * ohf, axis=1, keepdims=True)

    counts = running
    nb = jnp.floor((counts + (BT - 1)) / BT)
    u_r = lax.broadcasted_iota(jnp.int32, (E, E), 0)
    u_c = lax.broadcasted_iota(jnp.int32, (E, E), 1)
    u_incl = (u_r <= u_c).astype(jnp.float32)
    ic = lax.dot_general(
        nb, u_incl, (((1,), (0,)), ((), ())),
        preferred_element_type=jnp.float32)
    offs = (ic - nb) * BT
    pos = jnp.sum(ohf * offs, axis=1, keepdims=True) + rank
    pos_ref[...] = pos.astype(jnp.int32)

    b_iota = lax.broadcasted_iota(jnp.int32, (NBPAD, E), 0).astype(jnp.float32)
    be = jnp.sum((jnp.broadcast_to(ic, (NBPAD, E)) <= b_iota)
                 .astype(jnp.int32), axis=1, keepdims=True)
    be = jnp.minimum(be, E - 1)
    be_ref[...] = be
    used_ref[...] = ic[:, E - 1:E].astype(jnp.int32)

    present = (counts > 0.0).astype(jnp.float32)
    u_strict = (u_r < u_c).astype(jnp.float32)
    ordrank = lax.dot_general(
        present, u_strict, (((1,), (0,)), ((), ())),
        preferred_element_type=jnp.float32)
    slot_row = ordrank - 2.0 * jnp.floor(ordrank * 0.5)
    nxt_col = jnp.min(
        jnp.where((u_c > u_r) & (jnp.broadcast_to(present, (E, E)) > 0.0),
                  u_c.astype(jnp.float32), float(E)),
        axis=1, keepdims=True)
    eye = (u_r == u_c).astype(jnp.float32)
    nxt_row = lax.dot_general(
        nxt_col, eye, (((0,), (0,)), ((), ())),
        preferred_element_type=jnp.float32)
    e_iota_nb = lax.broadcasted_iota(jnp.int32, (NBPAD, E), 1)
    ohb = (e_iota_nb == be).astype(jnp.float32)
    sl_ref[...] = jnp.sum(ohb * slot_row, axis=1, keepdims=True).astype(jnp.int32)
    nxt_ref[...] = jnp.sum(ohb * nxt_row, axis=1, keepdims=True).astype(jnp.int32)


_router = pl.pallas_call(
    _router_body,
    out_shape=(
        jax.ShapeDtypeStruct((T, 1), jnp.int32),
        jax.ShapeDtypeStruct((T, 1), jnp.float32),
        jax.ShapeDtypeStruct((NBPAD, 1), jnp.int32),
        jax.ShapeDtypeStruct((1, 1), jnp.int32),
        jax.ShapeDtypeStruct((NBPAD, 1), jnp.int32),
        jax.ShapeDtypeStruct((NBPAD, 1), jnp.int32),
    ),
)


@functools.lru_cache(maxsize=None)
def _sc_kernels():
    mesh = plsc.VectorSubcoreMesh(core_axis_name="c", subcore_axis_name="s")

    @functools.partial(
        pl.kernel,
        out_type=jax.ShapeDtypeStruct((NBUF, HIDDEN), jnp.float32),
        scratch_types=[
            pltpu.VMEM((TPW,), jnp.int32),
            pltpu.VMEM((TPW, HIDDEN), jnp.float32),
            pltpu.SemaphoreType.DMA,
        ],
        mesh=mesh,
    )
    def dispatch(x_hbm, pos_hbm, xs_hbm, idx_v, rows_v, sem):
        wid = lax.axis_index("s") * 2 + lax.axis_index("c")
        base = wid * TPW
        pltpu.sync_copy(pos_hbm.at[pl.ds(base, TPW)], idx_v)
        pltpu.sync_copy(x_hbm.at[pl.ds(base, TPW)], rows_v)
        pltpu.async_copy(rows_v, xs_hbm.at[idx_v], sem).wait()

    @functools.partial(
        pl.kernel,
        out_type=jax.ShapeDtypeStruct((T, HIDDEN), jnp.float32),
        scratch_types=[
            pltpu.VMEM((TPW,), jnp.int32),
            pltpu.VMEM((TPW, HIDDEN), jnp.float32),
            pltpu.SemaphoreType.DMA,
        ],
        mesh=mesh,
    )
    def combine(ys_hbm, pos_hbm, out_hbm, idx_v, rows_v, sem):
        wid = lax.axis_index("s") * 2 + lax.axis_index("c")
        base = wid * TPW
        pltpu.sync_copy(pos_hbm.at[pl.ds(base, TPW)], idx_v)
        pltpu.async_copy(ys_hbm.at[idx_v], rows_v, sem).wait()
        pltpu.sync_copy(rows_v, out_hbm.at[pl.ds(base, TPW)])

    return dispatch, combine


HALF = INTER // 2


def _fetch_copies(e, s, wg_hbm, wu_hbm, wd_hbm, wg2, wu2, wd2, sem):
    return [
        pltpu.make_async_copy(h.at[e, pl.ds(o, HALF)],
                              v.at[s, pl.ds(o, HALF)], sem.at[s])
        for h, v in ((wg_hbm, wg2), (wu_hbm, wu2), (wd_hbm, wd2))
        for o in (0, HALF)
    ]


def _start_fetch(e, s, wg_hbm, wu_hbm, wd_hbm, wg2, wu2, wd2, sem):
    for c in _fetch_copies(e, s, wg_hbm, wu_hbm, wd_hbm, wg2, wu2, wd2, sem):
        c.start()


def _wait_fetch(e, s, wg_hbm, wu_hbm, wd_hbm, wg2, wu2, wd2, sem):
    for c in _fetch_copies(e, s, wg_hbm, wu_hbm, wd_hbm, wg2, wu2, wd2, sem):
        c.wait()


def _expert_body(be_ref, used_ref, sl_ref, nxt_ref, xs_ref,
                 wg_hbm, wu_hbm, wd_hbm, ys_ref,
                 wg2, wu2, wd2, sem):
    b = pl.program_id(0)
    used = used_ref[0]
    live = b < used
    e = be_ref[b]
    first = jnp.logical_and(
        live, jnp.logical_or(b == 0, e != be_ref[jnp.maximum(b - 1, 0)]))
    s = sl_ref[b]
    nx = nxt_ref[b]
    hbm = (wg_hbm, wu_hbm, wd_hbm)
    bufs = (wg2, wu2, wd2)

    @pl.when(b == 0)
    def _prime():
        _start_fetch(e, 0, *hbm, *bufs, sem)

    @pl.when(jnp.logical_and(first, nx < E))
    def _prefetch_next():
        @pl.when(s == 0)
        def _():
            _start_fetch(nx, 1, *hbm, *bufs, sem)

        @pl.when(s == 1)
        def _():
            _start_fetch(nx, 0, *hbm, *bufs, sem)

    @pl.when(first)
    def _arrive():
        @pl.when(s == 0)
        def _():
            _wait_fetch(e, 0, *hbm, *bufs, sem)

        @pl.when(s == 1)
        def _():
            _wait_fetch(e, 1, *hbm, *bufs, sem)

    @pl.when(live)
    def _compute():
        xb = xs_ref[...]
        wg = wg2[s]
        wu = wu2[s]
        wd = wd2[s]
        g = lax.dot_general(xb, wg, (((1,), (1,)), ((), ())),
                            preferred_element_type=jnp.float32,
                            precision=lax.Precision.DEFAULT)
        u = lax.dot_general(xb, wu, (((1,), (1,)), ((), ())),
                            preferred_element_type=jnp.float32,
                            precision=lax.Precision.DEFAULT)
        h = (g / (1.0 + jnp.exp(-g))) * u
        ys_ref[...] = lax.dot_general(
            h, wd, (((1,), (1,)), ((), ())),
            preferred_element_type=jnp.float32,
            precision=lax.Precision.DEFAULT)


_experts = pl.pallas_call(
    _expert_body,
    grid_spec=pltpu.PrefetchScalarGridSpec(
        num_scalar_prefetch=4,
        grid=(NB,),
        in_specs=[
            pl.BlockSpec((BT, HIDDEN),
                         lambda b, be, used, sl, nxt: (jnp.minimum(b, used[0] - 1), 0)),
            pl.BlockSpec(memory_space=pltpu.MemorySpace.HBM),
            pl.BlockSpec(memory_space=pltpu.MemorySpace.HBM),
            pl.BlockSpec(memory_space=pltpu.MemorySpace.HBM),
        ],
        out_specs=pl.BlockSpec((BT, HIDDEN), lambda b, be, used, sl, nxt: (b, 0)),
        scratch_shapes=[
            pltpu.VMEM((2, INTER, HIDDEN), jnp.float32),
            pltpu.VMEM((2, INTER, HIDDEN), jnp.float32),
            pltpu.VMEM((2, HIDDEN, INTER), jnp.float32),
            pltpu.SemaphoreType.DMA((2,)),
        ],
    ),
    out_shape=jax.ShapeDtypeStruct((NBUF, HIDDEN), jnp.float32),
)


def _scale_body(y_ref, p_ref, o_ref):
    o_ref[...] = y_ref[...] * p_ref[...]


_scale = pl.pallas_call(
    _scale_body,
    grid=(T // BT,),
    in_specs=[
        pl.BlockSpec((BT, HIDDEN), lambda i: (i, 0)),
        pl.BlockSpec((BT, 1), lambda i: (i, 0)),
    ],
    out_specs=pl.BlockSpec((BT, HIDDEN), lambda i: (i, 0)),
    out_shape=jax.ShapeDtypeStruct((T, HIDDEN), jnp.float32),
)


def kernel(x, gate_w, w_gate_proj, w_up_proj, w_down_proj):
    pos2d, pval2d, be2d, used2d, sl2d, nxt2d = _router(x, gate_w)
    pos = pos2d.reshape(T)
    be = be2d.reshape(NBPAD)
    used = used2d.reshape(1)
    sl = sl2d.reshape(NBPAD)
    nxt = nxt2d.reshape(NBPAD)
    dispatch, combine = _sc_kernels()
    xs = dispatch(x, pos)
    ys = _experts(be, used, sl, nxt, xs, w_gate_proj, w_up_proj, w_down_proj)
    yt = combine(ys, pos)
    return _scale(yt, pval2d)
